# baseline (speedup 1.0000x reference)
import functools

import jax
import jax.numpy as jnp
from jax.experimental import pallas as pl
from jax.experimental.pallas import tpu as pltpu


def _xw_kernel(x_ref, w_ref, xw_ref, xw2_ref):
    xw = jnp.dot(x_ref[...], w_ref[...], preferred_element_type=jnp.float32)
    xw_ref[...] = xw
    xw2_ref[...] = xw * xw


def _fused_kernel(a_ref, xw_ref, xw2_ref, xws_ref, xw2s_ref, bias_ref,
                  out_ref, *, bm, n):
    i = pl.program_id(0)

    a = a_ref[...]
    a2 = a * a

    s = jnp.dot(a, xw_ref[...], preferred_element_type=jnp.float32)
    t = jnp.dot(a2, xw2_ref[...], preferred_element_type=jnp.float32)
    rs = jnp.sum(a, axis=1, keepdims=True)
    rss = jnp.sum(a2, axis=1, keepdims=True)

    rows = i * bm + jax.lax.broadcasted_iota(jnp.int32, (bm, n), 0)
    cols = jax.lax.broadcasted_iota(jnp.int32, (bm, n), 1)
    adiag = jnp.sum(jnp.where(rows == cols, a, 0.0), axis=1, keepdims=True)

    c = 2.0 * adiag + 1.0
    s = s + xws_ref[...]
    t = t + c * xw2s_ref[...]
    rs = rs + 1.0
    rss = rss + c

    denom = rs * rs - rss
    inv = 1.0 / denom
    inv = jnp.where(jnp.isinf(inv), 0.0, inv)
    out_ref[...] = inv * (s * s - t) + bias_ref[...]


def _pick_block(n, cap):
    best = 1
    for d in range(1, n + 1):
        if n % d == 0 and d <= cap and d % 8 == 0:
            best = d
    return best if n % 8 == 0 else n


def kernel(x, edge_index, edge_weight, weight, bias):
    del edge_weight
    n, d_in = x.shape
    d_out = weight.shape[1]

    xw, xw2 = pl.pallas_call(
        _xw_kernel,
        out_shape=(
            jax.ShapeDtypeStruct((n, d_out), jnp.float32),
            jax.ShapeDtypeStruct((n, d_out), jnp.float32),
        ),
    )(x, weight)

    bm = _pick_block(n, 400)
    grid = (n // bm,)

    out = pl.pallas_call(
        functools.partial(_fused_kernel, bm=bm, n=n),
        grid=grid,
        in_specs=[
            pl.BlockSpec((bm, n), lambda i: (i, 0)),
            pl.BlockSpec((n, d_out), lambda i: (0, 0)),
            pl.BlockSpec((n, d_out), lambda i: (0, 0)),
            pl.BlockSpec((bm, d_out), lambda i: (i, 0)),
            pl.BlockSpec((bm, d_out), lambda i: (i, 0)),
            pl.BlockSpec((1, d_out), lambda i: (0, 0)),
        ],
        out_specs=pl.BlockSpec((bm, d_out), lambda i: (i, 0)),
        out_shape=jax.ShapeDtypeStruct((n, d_out), jnp.float32),
        compiler_params=pltpu.CompilerParams(
            dimension_semantics=("arbitrary",),
        ),
    )(edge_index, xw, xw2, xw, xw2, bias.reshape(1, d_out))

    return out


# speedup vs baseline: 1.1884x; 1.1884x over previous
import functools

import jax
import jax.numpy as jnp
from jax.experimental import pallas as pl
from jax.experimental.pallas import tpu as pltpu


def _xw_kernel(x_ref, w_ref, xw_ref, xw2_ref):
    xw = jnp.dot(x_ref[...], w_ref[...], preferred_element_type=jnp.float32)
    xw_ref[...] = xw.astype(jnp.bfloat16)
    xw2_ref[...] = (xw * xw).astype(jnp.bfloat16)


def _fused_kernel(a_ref, xw_ref, xw2_ref, bias_ref, out_ref, *, bm, n):
    i = pl.program_id(0)

    a = a_ref[...]
    rows = i * bm + jax.lax.broadcasted_iota(jnp.int32, (bm, n), 0)
    cols = jax.lax.broadcasted_iota(jnp.int32, (bm, n), 1)
    a = a + jnp.where(rows == cols, 1.0, 0.0).astype(a.dtype)
    a2 = a * a

    s = jnp.dot(a.astype(jnp.bfloat16), xw_ref[...],
                preferred_element_type=jnp.float32)
    t = jnp.dot(a2.astype(jnp.bfloat16), xw2_ref[...],
                preferred_element_type=jnp.float32)
    rs = jnp.sum(a, axis=1, keepdims=True)
    rss = jnp.sum(a2, axis=1, keepdims=True)

    denom = rs * rs - rss
    inv = 1.0 / denom
    inv = jnp.where(jnp.isinf(inv), 0.0, inv)
    out_ref[...] = inv * (s * s - t) + bias_ref[...]


def _pick_block(n, cap):
    best = 1
    for d in range(1, n + 1):
        if n % d == 0 and d <= cap and d % 8 == 0:
            best = d
    return best if n % 8 == 0 else n


def kernel(x, edge_index, edge_weight, weight, bias):
    del edge_weight
    n, d_in = x.shape
    d_out = weight.shape[1]

    xw, xw2 = pl.pallas_call(
        _xw_kernel,
        out_shape=(
            jax.ShapeDtypeStruct((n, d_out), jnp.bfloat16),
            jax.ShapeDtypeStruct((n, d_out), jnp.bfloat16),
        ),
    )(x, weight)

    bm = _pick_block(n, 400)
    grid = (n // bm,)

    out = pl.pallas_call(
        functools.partial(_fused_kernel, bm=bm, n=n),
        grid=grid,
        in_specs=[
            pl.BlockSpec((bm, n), lambda i: (i, 0)),
            pl.BlockSpec((n, d_out), lambda i: (0, 0)),
            pl.BlockSpec((n, d_out), lambda i: (0, 0)),
            pl.BlockSpec((1, d_out), lambda i: (0, 0)),
        ],
        out_specs=pl.BlockSpec((bm, d_out), lambda i: (i, 0)),
        out_shape=jax.ShapeDtypeStruct((n, d_out), jnp.float32),
        compiler_params=pltpu.CompilerParams(
            dimension_semantics=("arbitrary",),
        ),
    )(edge_index, xw, xw2, bias.reshape(1, d_out))

    return out


# speedup vs baseline: 1.2289x; 1.0340x over previous
import functools

import jax
import jax.numpy as jnp
from jax.experimental import pallas as pl
from jax.experimental.pallas import tpu as pltpu


def _fused_kernel(a_ref, x_ref, w_ref, bias_ref, out_ref, xw_ref, xw2_ref,
                  *, bm, n):
    i = pl.program_id(0)

    @pl.when(i == 0)
    def _compute_xw():
        xw = jnp.dot(x_ref[...], w_ref[...],
                     preferred_element_type=jnp.float32)
        xw_ref[...] = xw.astype(jnp.bfloat16)
        xw2_ref[...] = (xw * xw).astype(jnp.bfloat16)

    a = a_ref[...]
    rows = i * bm + jax.lax.broadcasted_iota(jnp.int32, (bm, n), 0)
    cols = jax.lax.broadcasted_iota(jnp.int32, (bm, n), 1)
    a = a + jnp.where(rows == cols, 1.0, 0.0).astype(a.dtype)
    a2 = a * a

    s = jnp.dot(a.astype(jnp.bfloat16), xw_ref[...],
                preferred_element_type=jnp.float32)
    t = jnp.dot(a2.astype(jnp.bfloat16), xw2_ref[...],
                preferred_element_type=jnp.float32)
    rs = jnp.sum(a, axis=1, keepdims=True)
    rss = jnp.sum(a2, axis=1, keepdims=True)

    denom = rs * rs - rss
    inv = 1.0 / denom
    inv = jnp.where(jnp.isinf(inv), 0.0, inv)
    out_ref[...] = inv * (s * s - t) + bias_ref[...]


def _pick_block(n, cap):
    best = 1
    for d in range(1, n + 1):
        if n % d == 0 and d <= cap and d % 8 == 0:
            best = d
    return best if n % 8 == 0 else n


def kernel(x, edge_index, edge_weight, weight, bias):
    del edge_weight
    n, d_in = x.shape
    d_out = weight.shape[1]

    bm = _pick_block(n, 400)
    grid = (n // bm,)

    out = pl.pallas_call(
        functools.partial(_fused_kernel, bm=bm, n=n),
        grid=grid,
        in_specs=[
            pl.BlockSpec((bm, n), lambda i: (i, 0)),
            pl.BlockSpec((n, d_in), lambda i: (0, 0)),
            pl.BlockSpec((d_in, d_out), lambda i: (0, 0)),
            pl.BlockSpec((1, d_out), lambda i: (0, 0)),
        ],
        out_specs=pl.BlockSpec((bm, d_out), lambda i: (i, 0)),
        out_shape=jax.ShapeDtypeStruct((n, d_out), jnp.float32),
        scratch_shapes=[
            pltpu.VMEM((n, d_out), jnp.bfloat16),
            pltpu.VMEM((n, d_out), jnp.bfloat16),
        ],
        compiler_params=pltpu.CompilerParams(
            dimension_semantics=("arbitrary",),
        ),
    )(edge_index, x, weight, bias.reshape(1, d_out))

    return out
